# baseline (speedup 1.0000x reference)
import functools

import jax
import jax.numpy as jnp
from jax import lax
from jax.experimental import pallas as pl
from jax.experimental.pallas import tpu as pltpu
from jax.experimental.pallas import tpu_sc as plsc

N = 10000
E = 320000
D = 128
NMOL = 4096
DEPTH = 3

BLK = 80
BLK_F = 256
S_ROUND = 10240
S_POOL = 4352

NW = 32
CHUNK = 128
G = 8


def _round_up(a, b):
    return -(-a // b) * b


CPT = _round_up(-(-E // (NW * CHUNK)), 2 * G)
CPT_P = _round_up(-(-N // (NW * CHUNK)), 2)


@functools.cache
def _make_sc_agg(s_pad: int, cpt: int, streamed: bool):
    rpt = s_pad // 16
    mesh = plsc.VectorSubcoreMesh(core_axis_name="c", subcore_axis_name="s",
                                  num_cores=2, num_subcores=16)
    if streamed:
        idx_scratch = [pltpu.VMEM((G, CHUNK), jnp.int32)] * 4
        n_idx_sems = 2
    else:
        idx_scratch = [pltpu.VMEM((cpt + 2, CHUNK), jnp.int32)] * 2
        n_idx_sems = 0

    @functools.partial(
        pl.kernel,
        mesh=mesh,
        out_type=jax.ShapeDtypeStruct((2 * s_pad, D), jnp.float32),
        scratch_types=[
            *idx_scratch,
            pltpu.VMEM((CHUNK, D), jnp.float32),
            pltpu.VMEM((CHUNK, D), jnp.float32),
            pltpu.VMEM_SHARED((s_pad, D), jnp.float32),
            pltpu.SemaphoreType.DMA,
            pltpu.SemaphoreType.DMA,
            *([pltpu.SemaphoreType.DMA] * n_idx_sems),
        ],
    )
    def sc_agg(feats_hbm, srcs_hbm, dsts_hbm, zeros_hbm, out_hbm, *rest):
        cid = lax.axis_index("c")
        sid = lax.axis_index("s")
        w = cid * 16 + sid

        if streamed:
            s0, s1, d0, d1, buf0, buf1, agg_s, gs0, gs1, is0, is1 = rest
            ngp2 = cpt // G + 2

            def fetch_group(gi, sbuf, dbuf, sem):
                pltpu.async_copy(srcs_hbm.at[w * ngp2 + gi], sbuf, sem)
                pltpu.async_copy(dsts_hbm.at[w * ngp2 + gi], dbuf, sem)

            def wait_group(sbuf, dbuf, sem):
                pltpu.make_async_copy(srcs_hbm.at[w * ngp2], sbuf, sem).wait()
                pltpu.make_async_copy(dsts_hbm.at[w * ngp2], dbuf, sem).wait()

            fetch_group(0, s0, d0, is0)
        else:
            src_v, dst_v, buf0, buf1, agg_s, gs0, gs1 = rest
            pltpu.sync_copy(srcs_hbm.at[w], src_v)
            pltpu.sync_copy(dsts_hbm.at[w], dst_v)

        pltpu.sync_copy(zeros_hbm.at[pl.ds(sid * rpt, rpt)],
                        agg_s.at[pl.ds(sid * rpt, rpt)])

        if streamed:
            wait_group(s0, d0, is0)
            fetch_group(1, s1, d1, is1)
            plsc.subcore_barrier()
            pltpu.async_copy(feats_hbm.at[s0.at[0]], buf0, gs0)
            pltpu.async_copy(feats_hbm.at[s0.at[1]], buf1, gs1)

            def body(i, carry):
                for m in range(2 * G):
                    buf, gs = (buf0, gs0) if m % 2 == 0 else (buf1, gs1)
                    here = s0.at[m] if m < G else s1.at[m - G]
                    dsts = d0.at[m] if m < G else d1.at[m - G]
                    pltpu.make_async_copy(feats_hbm.at[here], buf, gs).wait()
                    pltpu.sync_copy(buf, agg_s.at[dsts], add=True)
                    if m == G - 1:
                        fetch_group(2 * i + 2, s0, d0, is0)
                    if m == 2 * G - 1:
                        fetch_group(2 * i + 3, s1, d1, is1)
                    t = m + 2
                    if t == G:
                        wait_group(s1, d1, is1)
                    if t == 2 * G:
                        wait_group(s0, d0, is0)
                    if t < G:
                        nxt = s0.at[t]
                    elif t < 2 * G:
                        nxt = s1.at[t - G]
                    else:
                        nxt = s0.at[t - 2 * G]
                    pltpu.async_copy(feats_hbm.at[nxt], buf, gs)
                return carry

            lax.fori_loop(0, cpt // (2 * G), body, 0)
            pltpu.make_async_copy(feats_hbm.at[s0.at[0]], buf0, gs0).wait()
            pltpu.make_async_copy(feats_hbm.at[s0.at[1]], buf1, gs1).wait()
            wait_group(s1, d1, is1)
        else:
            plsc.subcore_barrier()
            pltpu.async_copy(feats_hbm.at[src_v.at[0]], buf0, gs0)
            pltpu.async_copy(feats_hbm.at[src_v.at[1]], buf1, gs1)

            def body(i, carry):
                c0 = 2 * i
                pltpu.make_async_copy(
                    feats_hbm.at[src_v.at[c0]], buf0, gs0).wait()
                pltpu.sync_copy(buf0, agg_s.at[dst_v.at[c0]], add=True)
                pltpu.async_copy(feats_hbm.at[src_v.at[c0 + 2]], buf0, gs0)
                pltpu.make_async_copy(
                    feats_hbm.at[src_v.at[c0 + 1]], buf1, gs1).wait()
                pltpu.sync_copy(buf1, agg_s.at[dst_v.at[c0 + 1]], add=True)
                pltpu.async_copy(feats_hbm.at[src_v.at[c0 + 3]], buf1, gs1)
                return carry

            lax.fori_loop(0, cpt // 2, body, 0)
            pltpu.make_async_copy(
                feats_hbm.at[src_v.at[cpt]], buf0, gs0).wait()
            pltpu.make_async_copy(
                feats_hbm.at[src_v.at[cpt + 1]], buf1, gs1).wait()

        plsc.subcore_barrier()
        pltpu.sync_copy(agg_s.at[pl.ds(sid * rpt, rpt)],
                        out_hbm.at[pl.ds(cid * s_pad + sid * rpt, rpt)])

    return sc_agg


def _pack_indices(src, dst, cpt, dump_base, streamed):
    n = src.shape[0]
    total = NW * cpt * CHUNK
    pad = total - n
    src_p = jnp.concatenate([src, jnp.zeros((pad,), jnp.int32)])
    dump = dump_base + (jnp.arange(pad, dtype=jnp.int32) % CHUNK)
    dst_p = jnp.concatenate([dst, dump])
    if streamed:
        ng = cpt // G
        over = jnp.zeros((NW, 2, G, CHUNK), jnp.int32)
        return tuple(
            jnp.concatenate([a.reshape(NW, ng, G, CHUNK), over], axis=1)
            .reshape(NW * (ng + 2), G, CHUNK)
            for a in (src_p, dst_p))
    over = jnp.zeros((NW, 2, CHUNK), jnp.int32)
    return (jnp.concatenate([src_p.reshape(NW, cpt, CHUNK), over], axis=1),
            jnp.concatenate([dst_p.reshape(NW, cpt, CHUNK), over], axis=1))


def _tc_h0(x_p, W_in):
    def body(x_ref, wi_ref, h0_ref):
        h0_ref[...] = jnp.maximum(
            jnp.dot(x_ref[...], wi_ref[...], preferred_element_type=jnp.float32),
            0.0)

    return pl.pallas_call(
        body,
        grid=(N // BLK,),
        in_specs=[
            pl.BlockSpec((BLK, D), lambda i: (i, 0)),
            pl.BlockSpec((D, D), lambda i: (0, 0)),
        ],
        out_specs=pl.BlockSpec((BLK, D), lambda i: (i, 0)),
        out_shape=jax.ShapeDtypeStruct((N, D), jnp.float32),
    )(x_p, W_in)


def _tc_round(h0, q, W_h):
    nb = N // BLK
    off = S_ROUND // BLK

    def body(h0_ref, q0_ref, q1_ref, wh_ref, h_ref):
        agg = q0_ref[...] + q1_ref[...]
        h_ref[...] = jnp.maximum(
            h0_ref[...]
            + jnp.dot(agg, wh_ref[...], preferred_element_type=jnp.float32),
            0.0)

    return pl.pallas_call(
        body,
        grid=(nb,),
        in_specs=[
            pl.BlockSpec((BLK, D), lambda i: (i, 0)),
            pl.BlockSpec((BLK, D), lambda i: (i, 0)),
            pl.BlockSpec((BLK, D), lambda i: (i + off, 0)),
            pl.BlockSpec((D, D), lambda i: (0, 0)),
        ],
        out_specs=pl.BlockSpec((BLK, D), lambda i: (i, 0)),
        out_shape=jax.ShapeDtypeStruct((N, D), jnp.float32),
    )(h0, q, q, W_h)


def _tc_final(m, W_ffn1, b_ffn1, W_out_p, b_out_p):
    nb = NMOL // BLK_F
    off = S_POOL // BLK_F

    def body(m0_ref, m1_ref, w1_ref, b1_ref, wo_ref, bo_ref, out_ref):
        mv = m0_ref[...] + m1_ref[...]
        z = jnp.maximum(
            jnp.dot(mv, w1_ref[...], preferred_element_type=jnp.float32)
            + b1_ref[...], 0.0)
        out_ref[...] = (
            jnp.dot(z, wo_ref[...], preferred_element_type=jnp.float32)
            + bo_ref[...])

    return pl.pallas_call(
        body,
        grid=(nb,),
        in_specs=[
            pl.BlockSpec((BLK_F, D), lambda i: (i, 0)),
            pl.BlockSpec((BLK_F, D), lambda i: (i + off, 0)),
            pl.BlockSpec((D, D), lambda i: (0, 0)),
            pl.BlockSpec((1, D), lambda i: (0, 0)),
            pl.BlockSpec((D, D), lambda i: (0, 0)),
            pl.BlockSpec((1, D), lambda i: (0, 0)),
        ],
        out_specs=pl.BlockSpec((BLK_F, D), lambda i: (i, 0)),
        out_shape=jax.ShapeDtypeStruct((NMOL, D), jnp.float32),
    )(m, m, W_ffn1, b_ffn1, W_out_p, b_out_p)


def kernel(x, edge_index, mol_ids, W_in, W_h, W_ffn1, b_ffn1, W_out, b_out):
    src = edge_index[0].astype(jnp.int32)
    dst = edge_index[1].astype(jnp.int32)

    srcs_r, dsts_r = _pack_indices(src, dst, CPT, N, True)
    srcs_p, dsts_p = _pack_indices(
        jnp.arange(N, dtype=jnp.int32), mol_ids.astype(jnp.int32),
        CPT_P, NMOL, False)
    zeros = jnp.zeros((S_ROUND, D), jnp.float32)

    h = _tc_h0(x, W_in)
    h0 = h
    sc_round = _make_sc_agg(S_ROUND, CPT, True)
    for _ in range(DEPTH):
        q = sc_round(h, srcs_r, dsts_r, zeros)
        h = _tc_round(h0, q, W_h)

    sc_pool = _make_sc_agg(S_POOL, CPT_P, False)
    m = sc_pool(h, srcs_p, dsts_p, zeros)

    W_out_p = jnp.pad(W_out, ((0, 0), (0, D - W_out.shape[1])))
    b_out_p = jnp.pad(b_out, (0, D - b_out.shape[0])).reshape(1, D)
    out_full = _tc_final(m, W_ffn1, b_ffn1.reshape(1, D), W_out_p, b_out_p)
    return out_full[:, :W_out.shape[1]]


# speedup vs baseline: 2.0984x; 2.0984x over previous
import functools

import jax
import jax.numpy as jnp
from jax import lax
from jax.experimental import pallas as pl
from jax.experimental.pallas import tpu as pltpu
from jax.experimental.pallas import tpu_sc as plsc

N = 10000
E = 320000
D = 128
NMOL = 4096
DEPTH = 3

BLK = 80
BLK_F = 256
S_ROUND = 10240
S_POOL = 4352

NW = 32
CHUNK = 128

CPT = -(-E // (NW * CHUNK))
CPT_P = -(-N // (NW * CHUNK))


@functools.cache
def _make_sc_agg(s_pad: int, cpt: int):
    rpt = s_pad // 16
    mesh = plsc.VectorSubcoreMesh(core_axis_name="c", subcore_axis_name="s",
                                  num_cores=2, num_subcores=16)

    @functools.partial(
        pl.kernel,
        mesh=mesh,
        out_type=jax.ShapeDtypeStruct((2 * s_pad, D), jnp.float32),
        scratch_types=[
            pltpu.VMEM((cpt, CHUNK), jnp.int32),
            pltpu.VMEM((cpt, CHUNK), jnp.int32),
            pltpu.VMEM((CHUNK, D), jnp.float32),
            pltpu.VMEM_SHARED((s_pad, D), jnp.float32),
            pltpu.SemaphoreType.DMA,
        ],
    )
    def sc_agg(feats_hbm, srcs_hbm, dsts_hbm, zeros_hbm, out_hbm,
               src_v, dst_v, buf, agg_s, gs):
        cid = lax.axis_index("c")
        sid = lax.axis_index("s")
        w = cid * 16 + sid
        pltpu.sync_copy(srcs_hbm.at[w], src_v)
        pltpu.sync_copy(dsts_hbm.at[w], dst_v)
        pltpu.sync_copy(zeros_hbm.at[pl.ds(sid * rpt, rpt)],
                        agg_s.at[pl.ds(sid * rpt, rpt)])
        plsc.subcore_barrier()

        def body(j, carry):
            pltpu.async_copy(feats_hbm.at[src_v.at[j]], buf, gs).wait()
            pltpu.sync_copy(buf, agg_s.at[dst_v.at[j]], add=True)
            return carry

        lax.fori_loop(0, cpt, body, 0)
        plsc.subcore_barrier()
        pltpu.sync_copy(agg_s.at[pl.ds(sid * rpt, rpt)],
                        out_hbm.at[pl.ds(cid * s_pad + sid * rpt, rpt)])

    return sc_agg


def _pack_indices(src, dst, cpt, dump_base):
    n = src.shape[0]
    total = NW * cpt * CHUNK
    pad = total - n
    src_p = jnp.concatenate([src, jnp.zeros((pad,), jnp.int32)])
    dump = dump_base + (jnp.arange(pad, dtype=jnp.int32) % CHUNK)
    dst_p = jnp.concatenate([dst, dump])
    return (src_p.reshape(NW, cpt, CHUNK), dst_p.reshape(NW, cpt, CHUNK))


def _tc_h0(x_p, W_in):
    def body(x_ref, wi_ref, h0_ref):
        h0_ref[...] = jnp.maximum(
            jnp.dot(x_ref[...], wi_ref[...], preferred_element_type=jnp.float32),
            0.0)

    return pl.pallas_call(
        body,
        grid=(N // BLK,),
        in_specs=[
            pl.BlockSpec((BLK, D), lambda i: (i, 0)),
            pl.BlockSpec((D, D), lambda i: (0, 0)),
        ],
        out_specs=pl.BlockSpec((BLK, D), lambda i: (i, 0)),
        out_shape=jax.ShapeDtypeStruct((N, D), jnp.float32),
    )(x_p, W_in)


def _tc_round(h0, q, W_h):
    nb = N // BLK
    off = S_ROUND // BLK

    def body(h0_ref, q0_ref, q1_ref, wh_ref, h_ref):
        agg = q0_ref[...] + q1_ref[...]
        h_ref[...] = jnp.maximum(
            h0_ref[...]
            + jnp.dot(agg, wh_ref[...], preferred_element_type=jnp.float32),
            0.0)

    return pl.pallas_call(
        body,
        grid=(nb,),
        in_specs=[
            pl.BlockSpec((BLK, D), lambda i: (i, 0)),
            pl.BlockSpec((BLK, D), lambda i: (i, 0)),
            pl.BlockSpec((BLK, D), lambda i: (i + off, 0)),
            pl.BlockSpec((D, D), lambda i: (0, 0)),
        ],
        out_specs=pl.BlockSpec((BLK, D), lambda i: (i, 0)),
        out_shape=jax.ShapeDtypeStruct((N, D), jnp.float32),
    )(h0, q, q, W_h)


def _tc_final(m, W_ffn1, b_ffn1, W_out_p, b_out_p):
    nb = NMOL // BLK_F
    off = S_POOL // BLK_F

    def body(m0_ref, m1_ref, w1_ref, b1_ref, wo_ref, bo_ref, out_ref):
        mv = m0_ref[...] + m1_ref[...]
        z = jnp.maximum(
            jnp.dot(mv, w1_ref[...], preferred_element_type=jnp.float32)
            + b1_ref[...], 0.0)
        out_ref[...] = (
            jnp.dot(z, wo_ref[...], preferred_element_type=jnp.float32)
            + bo_ref[...])

    return pl.pallas_call(
        body,
        grid=(nb,),
        in_specs=[
            pl.BlockSpec((BLK_F, D), lambda i: (i, 0)),
            pl.BlockSpec((BLK_F, D), lambda i: (i + off, 0)),
            pl.BlockSpec((D, D), lambda i: (0, 0)),
            pl.BlockSpec((1, D), lambda i: (0, 0)),
            pl.BlockSpec((D, D), lambda i: (0, 0)),
            pl.BlockSpec((1, D), lambda i: (0, 0)),
        ],
        out_specs=pl.BlockSpec((BLK_F, D), lambda i: (i, 0)),
        out_shape=jax.ShapeDtypeStruct((NMOL, D), jnp.float32),
    )(m, m, W_ffn1, b_ffn1, W_out_p, b_out_p)


def kernel(x, edge_index, mol_ids, W_in, W_h, W_ffn1, b_ffn1, W_out, b_out):
    src = edge_index[0].astype(jnp.int32)
    dst = edge_index[1].astype(jnp.int32)

    srcs_r, dsts_r = _pack_indices(src, dst, CPT, N)
    srcs_p, dsts_p = _pack_indices(
        jnp.arange(N, dtype=jnp.int32), mol_ids.astype(jnp.int32),
        CPT_P, NMOL)
    zeros = jnp.zeros((S_ROUND, D), jnp.float32)

    h = _tc_h0(x, W_in)
    h0 = h
    sc_round = _make_sc_agg(S_ROUND, CPT)
    for _ in range(DEPTH):
        q = sc_round(h, srcs_r, dsts_r, zeros)
        h = _tc_round(h0, q, W_h)

    sc_pool = _make_sc_agg(S_POOL, CPT_P)
    m = sc_pool(h, srcs_p, dsts_p, zeros)

    W_out_p = jnp.pad(W_out, ((0, 0), (0, D - W_out.shape[1])))
    b_out_p = jnp.pad(b_out, (0, D - b_out.shape[0])).reshape(1, D)
    out_full = _tc_final(m, W_ffn1, b_ffn1.reshape(1, D), W_out_p, b_out_p)
    return out_full[:, :W_out.shape[1]]


# speedup vs baseline: 2.6028x; 1.2404x over previous
import functools

import jax
import jax.numpy as jnp
from jax import lax
from jax.experimental import pallas as pl
from jax.experimental.pallas import tpu as pltpu
from jax.experimental.pallas import tpu_sc as plsc

N = 10000
E = 320000
D = 128
NMOL = 4096
DEPTH = 3

BLK = 512
BLK_F = 256
S_ROUND = 10240
S_POOL = 4352

NW = 32
CHUNK = 128

CPT = -(-E // (NW * CHUNK))
CPT_P = -(-N // (NW * CHUNK))


@functools.cache
def _make_sc_agg(s_pad: int, cpt: int):
    rpt = s_pad // 16
    mesh = plsc.VectorSubcoreMesh(core_axis_name="c", subcore_axis_name="s",
                                  num_cores=2, num_subcores=16)

    @functools.partial(
        pl.kernel,
        mesh=mesh,
        out_type=jax.ShapeDtypeStruct((2 * s_pad, D), jnp.float32),
        scratch_types=[
            pltpu.VMEM((cpt, CHUNK), jnp.int32),
            pltpu.VMEM((cpt, CHUNK), jnp.int32),
            pltpu.VMEM((CHUNK, D), jnp.float32),
            pltpu.VMEM_SHARED((s_pad, D), jnp.float32),
            pltpu.SemaphoreType.DMA,
        ],
    )
    def sc_agg(feats_hbm, srcs_hbm, dsts_hbm, zeros_hbm, out_hbm,
               src_v, dst_v, buf, agg_s, gs):
        cid = lax.axis_index("c")
        sid = lax.axis_index("s")
        w = cid * 16 + sid
        pltpu.sync_copy(srcs_hbm.at[w], src_v)
        pltpu.sync_copy(dsts_hbm.at[w], dst_v)
        pltpu.sync_copy(zeros_hbm.at[pl.ds(sid * rpt, rpt)],
                        agg_s.at[pl.ds(sid * rpt, rpt)])
        plsc.subcore_barrier()

        def body(j, carry):
            pltpu.async_copy(feats_hbm.at[src_v.at[j]], buf, gs).wait()
            pltpu.sync_copy(buf, agg_s.at[dst_v.at[j]], add=True)
            return carry

        lax.fori_loop(0, cpt, body, 0)
        plsc.subcore_barrier()
        pltpu.sync_copy(agg_s.at[pl.ds(sid * rpt, rpt)],
                        out_hbm.at[pl.ds(cid * s_pad + sid * rpt, rpt)])

    return sc_agg


def _pack_indices(src, dst, cpt, dump_base):
    n = src.shape[0]
    total = NW * cpt * CHUNK
    pad = total - n
    src_p = jnp.concatenate([src, jnp.zeros((pad,), jnp.int32)])
    dump = dump_base + (jnp.arange(pad, dtype=jnp.int32) % CHUNK)
    dst_p = jnp.concatenate([dst, dump])
    return (src_p.reshape(NW, cpt, CHUNK), dst_p.reshape(NW, cpt, CHUNK))


def _tc_h0(x_p, W_in):
    def body(x_ref, wi_ref, h0_ref):
        h0_ref[...] = jnp.maximum(
            jnp.dot(x_ref[...], wi_ref[...], preferred_element_type=jnp.float32),
            0.0)

    return pl.pallas_call(
        body,
        grid=(S_ROUND // BLK,),
        in_specs=[
            pl.BlockSpec((BLK, D), lambda i: (i, 0)),
            pl.BlockSpec((D, D), lambda i: (0, 0)),
        ],
        out_specs=pl.BlockSpec((BLK, D), lambda i: (i, 0)),
        out_shape=jax.ShapeDtypeStruct((S_ROUND, D), jnp.float32),
    )(x_p, W_in)


def _tc_round(h0, q, W_h):
    nb = S_ROUND // BLK
    off = S_ROUND // BLK

    def body(h0_ref, q0_ref, q1_ref, wh_ref, h_ref):
        agg = q0_ref[...] + q1_ref[...]
        h_ref[...] = jnp.maximum(
            h0_ref[...]
            + jnp.dot(agg, wh_ref[...], preferred_element_type=jnp.float32),
            0.0)

    return pl.pallas_call(
        body,
        grid=(nb,),
        in_specs=[
            pl.BlockSpec((BLK, D), lambda i: (i, 0)),
            pl.BlockSpec((BLK, D), lambda i: (i, 0)),
            pl.BlockSpec((BLK, D), lambda i: (i + off, 0)),
            pl.BlockSpec((D, D), lambda i: (0, 0)),
        ],
        out_specs=pl.BlockSpec((BLK, D), lambda i: (i, 0)),
        out_shape=jax.ShapeDtypeStruct((S_ROUND, D), jnp.float32),
    )(h0, q, q, W_h)


def _tc_final(m, W_ffn1, b_ffn1, W_out_p, b_out_p):
    nb = NMOL // BLK_F
    off = S_POOL // BLK_F

    def body(m0_ref, m1_ref, w1_ref, b1_ref, wo_ref, bo_ref, out_ref):
        mv = m0_ref[...] + m1_ref[...]
        z = jnp.maximum(
            jnp.dot(mv, w1_ref[...], preferred_element_type=jnp.float32)
            + b1_ref[...], 0.0)
        out_ref[...] = (
            jnp.dot(z, wo_ref[...], preferred_element_type=jnp.float32)
            + bo_ref[...])

    return pl.pallas_call(
        body,
        grid=(nb,),
        in_specs=[
            pl.BlockSpec((BLK_F, D), lambda i: (i, 0)),
            pl.BlockSpec((BLK_F, D), lambda i: (i + off, 0)),
            pl.BlockSpec((D, D), lambda i: (0, 0)),
            pl.BlockSpec((1, D), lambda i: (0, 0)),
            pl.BlockSpec((D, D), lambda i: (0, 0)),
            pl.BlockSpec((1, D), lambda i: (0, 0)),
        ],
        out_specs=pl.BlockSpec((BLK_F, D), lambda i: (i, 0)),
        out_shape=jax.ShapeDtypeStruct((NMOL, D), jnp.float32),
    )(m, m, W_ffn1, b_ffn1, W_out_p, b_out_p)


def kernel(x, edge_index, mol_ids, W_in, W_h, W_ffn1, b_ffn1, W_out, b_out):
    src = edge_index[0].astype(jnp.int32)
    dst = edge_index[1].astype(jnp.int32)

    srcs_r, dsts_r = _pack_indices(src, dst, CPT, N)
    srcs_p, dsts_p = _pack_indices(
        jnp.arange(N, dtype=jnp.int32), mol_ids.astype(jnp.int32),
        CPT_P, NMOL)
    zeros = jnp.zeros((S_ROUND, D), jnp.float32)
    x_p = jnp.pad(x, ((0, S_ROUND - N), (0, 0)))

    h = _tc_h0(x_p, W_in)
    h0 = h
    sc_round = _make_sc_agg(S_ROUND, CPT)
    for _ in range(DEPTH):
        q = sc_round(h, srcs_r, dsts_r, zeros)
        h = _tc_round(h0, q, W_h)

    sc_pool = _make_sc_agg(S_POOL, CPT_P)
    m = sc_pool(h, srcs_p, dsts_p, zeros)

    W_out_p = jnp.pad(W_out, ((0, 0), (0, D - W_out.shape[1])))
    b_out_p = jnp.pad(b_out, (0, D - b_out.shape[0])).reshape(1, D)
    out_full = _tc_final(m, W_ffn1, b_ffn1.reshape(1, D), W_out_p, b_out_p)
    return out_full[:, :W_out.shape[1]]
